# baseline (speedup 1.0000x reference)
import functools

import jax
import jax.numpy as jnp
from jax import lax
from jax.experimental import pallas as pl
from jax.experimental.pallas import tpu as pltpu
from jax.experimental.pallas import tpu_sc as plsc

_B = 16384
_E = 16
_NC = 2
_NS = 16
_NW = _NC * _NS
_CHUNK = 128
_NCHUNK = _B // (_NW * _CHUNK)

_MLP_BLK = 2048


def _gather_body(idx_hbm, table_hbm, out_hbm, idx_v, rows_v, sem):
    wid = lax.axis_index("s") * _NC + lax.axis_index("c")
    pltpu.sync_copy(idx_hbm.at[wid], idx_v)
    copies = [
        pltpu.async_copy(table_hbm.at[idx_v.at[j]], rows_v.at[j], sem)
        for j in range(_NCHUNK)
    ]
    for c in copies:
        c.wait()
    pltpu.sync_copy(rows_v, out_hbm.at[wid])


_gather = functools.partial(
    pl.kernel,
    out_type=jax.ShapeDtypeStruct((_NW, _NCHUNK, _CHUNK, _E), jnp.float32),
    mesh=plsc.VectorSubcoreMesh(core_axis_name="c", subcore_axis_name="s"),
    scratch_types=[
        pltpu.VMEM((_NCHUNK, _CHUNK), jnp.int32),
        pltpu.VMEM((_NCHUNK, _CHUNK, _E), jnp.float32),
        pltpu.SemaphoreType.DMA,
    ],
    compiler_params=pltpu.CompilerParams(use_tc_tiling_on_sc=False),
)(_gather_body)


def _mlp_body(h_ref, w1_ref, b1_ref, w2_ref, b2_ref, w3_ref, b3_ref, o_ref):
    h = h_ref[...]
    h1 = jnp.maximum(
        jnp.dot(h, w1_ref[...], preferred_element_type=jnp.float32) + b1_ref[...],
        0.0,
    )
    h2 = jnp.maximum(
        jnp.dot(h1, w2_ref[...], preferred_element_type=jnp.float32) + b2_ref[...],
        0.0,
    )
    o = jnp.dot(h2, w3_ref[...], preferred_element_type=jnp.float32) + b3_ref[...]
    o_ref[...] = jax.nn.sigmoid(o)


def _mlp(h, w1, b1, w2, b2, w3, b3):
    grid = (_B // _MLP_BLK,)
    fixed = lambda shape: pl.BlockSpec(shape, lambda i: (0, 0))
    return pl.pallas_call(
        _mlp_body,
        grid=grid,
        in_specs=[
            pl.BlockSpec((_MLP_BLK, _E), lambda i: (i, 0)),
            fixed((16, 32)),
            fixed((1, 32)),
            fixed((32, 16)),
            fixed((1, 16)),
            fixed((16, 1)),
            fixed((1, 1)),
        ],
        out_specs=pl.BlockSpec((_MLP_BLK, 1), lambda i: (i, 0)),
        out_shape=jax.ShapeDtypeStruct((_B, 1), jnp.float32),
    )(h, w1, b1, w2, b2, w3, b3)


def kernel(x_word, table, W1, b1, W2, b2, W3, b3):
    idx = x_word.astype(jnp.int32).reshape(_NW, _NCHUNK, _CHUNK)
    rows = _gather(idx, table)
    h = rows.reshape(_B, _E)
    out = _mlp(
        h,
        W1.T,
        b1.reshape(1, 32),
        W2.T,
        b2.reshape(1, 16),
        W3.T,
        b3.reshape(1, 1),
    )
    return out.reshape(_B)
